# baseline (speedup 1.0000x reference)
import functools
import jax
import jax.numpy as jnp
from jax import lax
from jax.experimental import pallas as pl
from jax.experimental.pallas import tpu as pltpu
from jax.experimental.pallas import tpu_sc as plsc

NREL = 1000
DW = 128 * 128
DB = 8 * 128
B = 1024

NC = 2
NS = 16
NW = NC * NS
BPW = B // NW
G = 2
NCHUNK = BPW // G


def _gather_body(rel_hbm, rel2_hbm, w_hbm, b_hbm, w_out, b_out,
                 idx2, idxb, wbuf0, wbuf1, bbuf, sem0, sem1, semb):
    cid = lax.axis_index("c")
    sid = lax.axis_index("s")
    wid = sid * NC + cid
    base = wid * BPW

    pltpu.sync_copy(rel2_hbm.at[pl.ds(wid * NCHUNK, NCHUNK)], idx2)
    pltpu.sync_copy(rel_hbm.at[pl.ds(base, BPW)], idxb)

    bias_cp = pltpu.async_copy(b_hbm.at[idxb], bbuf, semb)

    cp0 = pltpu.async_copy(w_hbm.at[idx2.at[0]], wbuf0, sem0)
    cp1 = pltpu.async_copy(w_hbm.at[idx2.at[1]], wbuf1, sem1)
    copies = [cp0, cp1]
    bufs = [wbuf0, wbuf1]
    sems = [sem0, sem1]
    for j in range(NCHUNK):
        bsel = j % 2
        copies[bsel].wait()
        pltpu.sync_copy(bufs[bsel], w_out.at[pl.ds(base + j * G, G)])
        if j + 2 < NCHUNK:
            copies[bsel] = pltpu.async_copy(
                w_hbm.at[idx2.at[j + 2]], bufs[bsel], sems[bsel])

    bias_cp.wait()
    pltpu.sync_copy(bbuf, b_out.at[pl.ds(base, BPW)])


@jax.jit
def kernel(relation, mlp_weight, mlp_bias):
    w2 = mlp_weight.reshape(NREL, DW)
    b2 = mlp_bias.reshape(NREL, DB)
    rel2 = relation.reshape(NW * NCHUNK, G)

    k = pl.kernel(
        _gather_body,
        out_type=[
            jax.ShapeDtypeStruct((B, DW), jnp.float32),
            jax.ShapeDtypeStruct((B, DB), jnp.float32),
        ],
        mesh=plsc.VectorSubcoreMesh(core_axis_name="c", subcore_axis_name="s"),
        scratch_types=[
            pltpu.VMEM((NCHUNK, G), jnp.int32),
            pltpu.VMEM((BPW,), jnp.int32),
            pltpu.VMEM((G, DW), jnp.float32),
            pltpu.VMEM((G, DW), jnp.float32),
            pltpu.VMEM((BPW, DB), jnp.float32),
            pltpu.SemaphoreType.DMA,
            pltpu.SemaphoreType.DMA,
            pltpu.SemaphoreType.DMA,
        ],
    )
    w_flat, b_flat = k(relation, rel2, w2, b2)
    return (w_flat.reshape(B, 128, 128), b_flat.reshape(B, 8, 128))


# speedup vs baseline: 2.4479x; 2.4479x over previous
import functools
import jax
import jax.numpy as jnp
from jax import lax
from jax.experimental import pallas as pl
from jax.experimental.pallas import tpu as pltpu
from jax.experimental.pallas import tpu_sc as plsc

NREL = 1000
B = 1024

NC = 2
NS = 16
NW = NC * NS
BPW = B // NW
G = 2
NCHUNK = BPW // G


def _gather_body(rel_hbm, rel2_hbm, w_hbm, b_hbm, w_out, b_out,
                 idx2, idxb, wbuf0, wbuf1, bbuf, sem0, sem1, semb):
    cid = lax.axis_index("c")
    sid = lax.axis_index("s")
    wid = sid * NC + cid
    base = wid * BPW

    pltpu.sync_copy(rel2_hbm.at[pl.ds(wid * NCHUNK, NCHUNK)], idx2)
    pltpu.sync_copy(rel_hbm.at[pl.ds(base, BPW)], idxb)

    bias_cp = pltpu.async_copy(b_hbm.at[idxb], bbuf, semb)

    cp0 = pltpu.async_copy(w_hbm.at[idx2.at[0]], wbuf0, sem0)
    cp1 = pltpu.async_copy(w_hbm.at[idx2.at[1]], wbuf1, sem1)
    copies = [cp0, cp1]
    bufs = [wbuf0, wbuf1]
    sems = [sem0, sem1]
    for j in range(NCHUNK):
        bsel = j % 2
        copies[bsel].wait()
        pltpu.sync_copy(bufs[bsel], w_out.at[pl.ds(base + j * G, G)])
        if j + 2 < NCHUNK:
            copies[bsel] = pltpu.async_copy(
                w_hbm.at[idx2.at[j + 2]], bufs[bsel], sems[bsel])

    bias_cp.wait()
    pltpu.sync_copy(bbuf, b_out.at[pl.ds(base, BPW)])


@jax.jit
def kernel(relation, mlp_weight, mlp_bias):
    rel2 = relation.reshape(NW * NCHUNK, G)

    k = pl.kernel(
        _gather_body,
        out_type=[
            jax.ShapeDtypeStruct((B, 128, 128), jnp.float32),
            jax.ShapeDtypeStruct((B, 8, 128), jnp.float32),
        ],
        mesh=plsc.VectorSubcoreMesh(core_axis_name="c", subcore_axis_name="s"),
        scratch_types=[
            pltpu.VMEM((NCHUNK, G), jnp.int32),
            pltpu.VMEM((BPW,), jnp.int32),
            pltpu.VMEM((G, 128, 128), jnp.float32),
            pltpu.VMEM((G, 128, 128), jnp.float32),
            pltpu.VMEM((BPW, 8, 128), jnp.float32),
            pltpu.SemaphoreType.DMA,
            pltpu.SemaphoreType.DMA,
            pltpu.SemaphoreType.DMA,
        ],
    )
    return tuple(k(relation, rel2, mlp_weight, mlp_bias))
